# Initial kernel scaffold; baseline (speedup 1.0000x reference)
import functools

import jax
import jax.numpy as jnp
from jax import lax
from jax.experimental import pallas as pl
from jax.experimental.pallas import tpu as pltpu
from jax.experimental.pallas import tpu_sc as plsc

_N = 10000
_E = 320000
_D = 128
_K = 5
_NC, _NS, _L = 2, 16, 16
_NW = _NC * _NS
_EPT = _E // _NW
_B = 40
_NB = _EPT // _B
_NPAD = 10240
_RPT = _NPAD // _NS
_CH = _D // _L


def _make_edge_agg(k_eff: int, with_cnt: bool):
  tw = k_eff * _D
  mesh = plsc.VectorSubcoreMesh(core_axis_name="c", subcore_axis_name="s")
  out_type = [jax.ShapeDtypeStruct((_NC, _NPAD, _D), jnp.float32)]
  if with_cnt:
    out_type.append(jax.ShapeDtypeStruct((_NC, _NPAD, _L), jnp.float32))
  scratch = [
      pltpu.VMEM_SHARED((_NPAD, _D), jnp.float32),
      pltpu.VMEM((_B,), jnp.int32),
      pltpu.VMEM((_B,), jnp.int32),
      pltpu.VMEM((k_eff, _B), jnp.float32),
      pltpu.VMEM((_B, tw), jnp.float32),
      pltpu.VMEM((_B, _D), jnp.float32),
      pltpu.SemaphoreType.DMA,
  ]
  if with_cnt:
    scratch += [
        pltpu.VMEM_SHARED((_NPAD, _L), jnp.float32),
        pltpu.VMEM((_B, _L), jnp.float32),
    ]

  @functools.partial(pl.kernel, out_type=tuple(out_type), mesh=mesh,
                     scratch_types=tuple(scratch))
  def k(table, srcw, dstw, gsub, *rest):
    if with_cnt:
      agg_out, cnt_out = rest[0], rest[1]
      agg_sh, src_v, dst_v, gs_v, rows_v, msg_v, sem, cnt_sh, ones_v = rest[2:]
    else:
      agg_out = rest[0]
      agg_sh, src_v, dst_v, gs_v, rows_v, msg_v, sem = rest[1:]
    cid = lax.axis_index("c")
    sid = lax.axis_index("s")
    w0 = (cid * _NS + sid) * _EPT
    r0 = sid * _RPT

    zero16 = jnp.zeros((_L,), jnp.float32)

    def zmsg(j, carry):
      for c in range(_CH):
        msg_v[j, pl.ds(c * _L, _L)] = zero16
      return carry
    lax.fori_loop(0, _B, zmsg, 0)
    for r in range(_RPT // _B):
      pltpu.sync_copy(msg_v, agg_sh.at[pl.ds(r0 + r * _B, _B)])
    if with_cnt:
      def zone(j, carry):
        ones_v[j, pl.ds(0, _L)] = zero16
        return carry
      lax.fori_loop(0, _B, zone, 0)
      for r in range(_RPT // _B):
        pltpu.sync_copy(ones_v, cnt_sh.at[pl.ds(r0 + r * _B, _B)])
      one16 = jnp.ones((_L,), jnp.float32)
      def fone(j, carry):
        ones_v[j, pl.ds(0, _L)] = one16
        return carry
      lax.fori_loop(0, _B, fone, 0)
    plsc.subcore_barrier()

    def batch(b, carry):
      e0 = b * _B
      pltpu.sync_copy(srcw.at[cid, sid, pl.ds(e0, _B)], src_v)
      pltpu.sync_copy(dstw.at[cid, sid, pl.ds(e0, _B)], dst_v)
      pltpu.sync_copy(gsub.at[:, pl.ds(w0 + e0, _B)], gs_v)
      pltpu.async_copy(table.at[src_v], rows_v, sem).wait()

      def edge(j, ecarry):
        g = [gs_v[k, j] for k in range(k_eff)]
        for c in range(_CH):
          acc = g[0] * rows_v[j, pl.ds(c * _L, _L)]
          for kk in range(1, k_eff):
            acc = acc + g[kk] * rows_v[j, pl.ds(kk * _D + c * _L, _L)]
          msg_v[j, pl.ds(c * _L, _L)] = acc
        return ecarry
      lax.fori_loop(0, _B, edge, 0)

      pltpu.sync_copy(msg_v, agg_sh.at[dst_v], add=True)
      if with_cnt:
        pltpu.sync_copy(ones_v, cnt_sh.at[dst_v], add=True)
      return carry
    lax.fori_loop(0, _NB, batch, 0)

    plsc.subcore_barrier()
    pltpu.sync_copy(agg_sh.at[pl.ds(r0, _RPT)], agg_out.at[cid, pl.ds(r0, _RPT)])
    if with_cnt:
      pltpu.sync_copy(cnt_sh.at[pl.ds(r0, _RPT)],
                      cnt_out.at[cid, pl.ds(r0, _RPT)])

  return k


_edge_agg_k5_cnt = _make_edge_agg(_K, True)
_edge_agg_k5 = _make_edge_agg(_K, False)
_edge_agg_k1 = _make_edge_agg(1, False)


def _matmul(x, w):
  n, d = x.shape
  m = w.shape[1]
  br = 1000

  def body(x_ref, w_ref, o_ref):
    o_ref[...] = jnp.dot(x_ref[...], w_ref[...],
                         preferred_element_type=jnp.float32)

  return pl.pallas_call(
      body,
      grid=(n // br,),
      in_specs=[pl.BlockSpec((br, d), lambda i: (i, 0)),
                pl.BlockSpec((d, m), lambda i: (0, 0))],
      out_specs=pl.BlockSpec((br, m), lambda i: (i, 0)),
      out_shape=jax.ShapeDtypeStruct((n, m), jnp.float32),
  )(x, w)


def _gauss_t(ea_t8, w1, w2, c):
  be = 8000

  def body(e_ref, w1_ref, w2_ref, c_ref, o_ref):
    e = e_ref[...]
    qa = lax.dot_general(w2_ref[...], e * e, (((0,), (0,)), ((), ())),
                         preferred_element_type=jnp.float32)
    qb = lax.dot_general(w1_ref[...], e, (((0,), (0,)), ((), ())),
                         preferred_element_type=jnp.float32)
    o_ref[...] = jnp.exp(qa + qb + c_ref[...])

  return pl.pallas_call(
      body,
      grid=(_E // be,),
      in_specs=[pl.BlockSpec((8, be), lambda i: (0, i)),
                pl.BlockSpec((8, 16), lambda i: (0, 0)),
                pl.BlockSpec((8, 16), lambda i: (0, 0)),
                pl.BlockSpec((16, 1), lambda i: (0, 0))],
      out_specs=pl.BlockSpec((16, be), lambda i: (0, i)),
      out_shape=jax.ShapeDtypeStruct((16, _E), jnp.float32),
  )(ea_t8, w1, w2, c)


def _bn(y, gamma, beta):
  m = jnp.mean(y, axis=0, keepdims=True)
  v = jnp.mean((y - m) ** 2, axis=0, keepdims=True)
  return (y - m) * lax.rsqrt(v + 1e-5) * gamma + beta


def _elu(y):
  return jnp.where(y > 0, y, jnp.exp(jnp.minimum(y, 0.0)) - 1.0)


def _combine1(p0, p1, q0, q1, c0, c1, xr1, b1, g1, be1, xrs, bs, gsc, bes):
  full = lambda s: pl.BlockSpec(s, lambda: (0,) * len(s))

  def body(p0_r, p1_r, q0_r, q1_r, c0_r, c1_r, xr1_r, b1_r, g1_r, be1_r,
           xrs_r, bs_r, gsc_r, bes_r, h_ref, s_ref):
    cnt = jnp.clip(c0_r[...][:, :1] + c1_r[...][:, :1], 1.0)
    y = (p0_r[...] + p1_r[...]) / cnt + xr1_r[...] + b1_r[...]
    h_ref[...] = _elu(_bn(y, g1_r[...], be1_r[...]))
    ys = (q0_r[...] + q1_r[...]) / cnt + xrs_r[...] + bs_r[...]
    s_ref[...] = _bn(ys, gsc_r[...], bes_r[...])

  nd = (_N, _D)
  return pl.pallas_call(
      body,
      in_specs=[full(nd), full(nd), full(nd), full(nd),
                full((_N, _L)), full((_N, _L)),
                full(nd), full((1, _D)), full((1, _D)), full((1, _D)),
                full(nd), full((1, _D)), full((1, _D)), full((1, _D))],
      out_specs=(full(nd), full(nd)),
      out_shape=(jax.ShapeDtypeStruct(nd, jnp.float32),
                 jax.ShapeDtypeStruct(nd, jnp.float32)),
  )(p0, p1, q0, q1, c0, c1, xr1, b1, g1, be1, xrs, bs, gsc, bes)


def _combine2(p0, p1, c0, c1, xr2, b2, g2, be2, scbn):
  full = lambda s: pl.BlockSpec(s, lambda: (0,) * len(s))

  def body(p0_r, p1_r, c0_r, c1_r, xr2_r, b2_r, g2_r, be2_r, sc_r, o_ref):
    cnt = jnp.clip(c0_r[...][:, :1] + c1_r[...][:, :1], 1.0)
    y = (p0_r[...] + p1_r[...]) / cnt + xr2_r[...] + b2_r[...]
    o_ref[...] = _elu(_bn(y, g2_r[...], be2_r[...]) + sc_r[...])

  nd = (_N, _D)
  return pl.pallas_call(
      body,
      in_specs=[full(nd), full(nd), full((_N, _L)), full((_N, _L)),
                full(nd), full((1, _D)), full((1, _D)), full((1, _D)),
                full(nd)],
      out_specs=full(nd),
      out_shape=jax.ShapeDtypeStruct(nd, jnp.float32),
  )(p0, p1, c0, c1, xr2, b2, g2, be2, scbn)


def kernel(x, edge_index, edge_attr, g1, mu1, sigma1, root1, b1,
           g2, mu2, sigma2, root2, b2, gs, mus, sigmas, roots, bs,
           bn1_g, bn1_b, bn2_g, bn2_b, bns_g, bns_b):
  f32 = jnp.float32
  src = edge_index[0].reshape(_NC, _NS, _EPT)
  dst = edge_index[1].reshape(_NC, _NS, _EPT)

  mu_all = jnp.concatenate([mu1, mu2, mus, jnp.zeros((5, 3), f32)], axis=0)
  sig_all = jnp.concatenate([sigma1, sigma2, sigmas, jnp.ones((5, 3), f32)],
                            axis=0)
  s2 = 1e-14 + sig_all ** 2
  w2 = jnp.concatenate([(-0.5 / s2).T, jnp.zeros((5, 16), f32)], axis=0)
  w1 = jnp.concatenate([(mu_all / s2).T, jnp.zeros((5, 16), f32)], axis=0)
  cq = (-0.5 * mu_all ** 2 / s2).sum(axis=1).reshape(16, 1)
  ea_t = jnp.concatenate([edge_attr.T, jnp.zeros((5, _E), f32)], axis=0)
  gauss_t = _gauss_t(ea_t, w1, w2, cq)

  wcat = jnp.concatenate([g1, gs, root1, roots], axis=1)
  xcat = _matmul(x, wcat)
  xt1, xts = xcat[:, : _K * _D], xcat[:, _K * _D: _K * _D + _D]
  xr1, xrs = xcat[:, 768:896], xcat[:, 896:1024]

  agg1, cnt = _edge_agg_k5_cnt(xt1, src, dst, gauss_t[0:_K])
  aggs = _edge_agg_k1(xts, src, dst, gauss_t[10:11])[0]

  c0, c1 = cnt[0, :_N], cnt[1, :_N]
  row = lambda v: v.reshape(1, _D)
  h, scbn = _combine1(agg1[0, :_N], agg1[1, :_N], aggs[0, :_N], aggs[1, :_N],
                      c0, c1, xr1, row(b1), row(bn1_g), row(bn1_b),
                      xrs, row(bs), row(bns_g), row(bns_b))

  w2cat = jnp.concatenate([g2, root2], axis=1)
  hcat = _matmul(h, w2cat)
  xt2, xr2 = hcat[:, : _K * _D], hcat[:, _K * _D:]

  agg2 = _edge_agg_k5(xt2, src, dst, gauss_t[_K:2 * _K])[0]

  return _combine2(agg2[0, :_N], agg2[1, :_N], c0, c1, xr2,
                   row(b2), row(bn2_g), row(bn2_b), scbn)


# speedup vs baseline: 2.0598x; 2.0598x over previous
import functools

import jax
import jax.numpy as jnp
from jax import lax
from jax.experimental import pallas as pl
from jax.experimental.pallas import tpu as pltpu
from jax.experimental.pallas import tpu_sc as plsc

_N = 10000
_E = 320000
_D = 128
_K = 5
_NC, _NS, _L = 2, 16, 16
_NW = _NC * _NS
_EPT = _E // _NW
_B = 40
_NB = _EPT // _B
_NPAD = 10112
_RPT = _NPAD // _NS
_CH = _D // _L


def _make_edge_agg(k_eff: int, k0: int):
  tw = k_eff * _D
  mesh = plsc.VectorSubcoreMesh(core_axis_name="c", subcore_axis_name="s")
  out_type = jax.ShapeDtypeStruct((_NC, _NPAD, _D), jnp.float32)
  scratch = [
      pltpu.VMEM_SHARED((_NPAD, _D), jnp.float32),
      pltpu.VMEM((_B,), jnp.int32),
      pltpu.VMEM((_B,), jnp.int32),
      pltpu.VMEM((_B, _L), jnp.float32),
      pltpu.VMEM((_B, tw), jnp.float32),
      pltpu.VMEM((_B, _D), jnp.float32),
      pltpu.SemaphoreType.DMA,
  ]

  @functools.partial(pl.kernel, out_type=out_type, mesh=mesh,
                     scratch_types=tuple(scratch))
  def k(table, srcw, dstw, gsub, agg_out, agg_sh, src_v, dst_v, gs_v,
        rows_v, msg_v, sem):
    cid = lax.axis_index("c")
    sid = lax.axis_index("s")
    w0 = (cid * _NS + sid) * _EPT
    r0 = sid * _RPT

    zero16 = jnp.zeros((_L,), jnp.float32)

    def zmsg(j, carry):
      for c in range(_CH):
        msg_v[j, pl.ds(c * _L, _L)] = zero16
      return carry
    lax.fori_loop(0, _B, zmsg, 0)
    nfull, rem = _RPT // _B, _RPT % _B
    for r in range(nfull):
      pltpu.sync_copy(msg_v, agg_sh.at[pl.ds(r0 + r * _B, _B)])
    if rem:
      pltpu.sync_copy(msg_v.at[pl.ds(0, rem)],
                      agg_sh.at[pl.ds(r0 + nfull * _B, rem)])
    plsc.subcore_barrier()

    def batch(b, carry):
      e0 = b * _B
      pltpu.sync_copy(srcw.at[pl.ds(w0 + e0, _B)], src_v)
      pltpu.sync_copy(dstw.at[pl.ds(w0 + e0, _B)], dst_v)
      pltpu.sync_copy(gsub.at[pl.ds(w0 + e0, _B)], gs_v)
      pltpu.async_copy(table.at[src_v], rows_v, sem).wait()

      def edge(j, ecarry):
        gvec = gs_v[j, pl.ds(0, _L)]
        g = [gvec[k0 + k] for k in range(k_eff)]
        for c in range(_CH):
          acc = g[0] * rows_v[j, pl.ds(c * _L, _L)]
          for kk in range(1, k_eff):
            acc = acc + g[kk] * rows_v[j, pl.ds(kk * _D + c * _L, _L)]
          msg_v[j, pl.ds(c * _L, _L)] = acc
        return ecarry
      lax.fori_loop(0, _B, edge, 0)

      pltpu.sync_copy(msg_v, agg_sh.at[dst_v], add=True)
      return carry
    lax.fori_loop(0, _NB, batch, 0)

    plsc.subcore_barrier()
    pltpu.sync_copy(agg_sh.at[pl.ds(r0, _RPT)],
                    agg_out.at[cid, pl.ds(r0, _RPT)])

  return k


_edge_agg_k5a = _make_edge_agg(_K, 0)
_edge_agg_k5b = _make_edge_agg(_K, _K)
_edge_agg_k1 = _make_edge_agg(1, 2 * _K)


def _make_cnt():
  mesh = plsc.VectorSubcoreMesh(core_axis_name="c", subcore_axis_name="s")
  out_type = jax.ShapeDtypeStruct((_NC, _NPAD, _D), jnp.float32)
  scratch = [
      pltpu.VMEM_SHARED((_NPAD, _D), jnp.float32),
      pltpu.VMEM((_B,), jnp.int32),
      pltpu.VMEM((_B, _D), jnp.float32),
  ]

  @functools.partial(pl.kernel, out_type=out_type, mesh=mesh,
                     scratch_types=tuple(scratch))
  def k(dstw, cnt_out, cnt_sh, dst_v, ones_v):
    cid = lax.axis_index("c")
    sid = lax.axis_index("s")
    w0 = (cid * _NS + sid) * _EPT
    r0 = sid * _RPT

    zero16 = jnp.zeros((_L,), jnp.float32)
    def zloop(j, carry):
      for c in range(_CH):
        ones_v[j, pl.ds(c * _L, _L)] = zero16
      return carry
    lax.fori_loop(0, _B, zloop, 0)
    nfull, rem = _RPT // _B, _RPT % _B
    for r in range(nfull):
      pltpu.sync_copy(ones_v, cnt_sh.at[pl.ds(r0 + r * _B, _B)])
    if rem:
      pltpu.sync_copy(ones_v.at[pl.ds(0, rem)],
                      cnt_sh.at[pl.ds(r0 + nfull * _B, rem)])
    one16 = jnp.ones((_L,), jnp.float32)
    def oloop(j, carry):
      for c in range(_CH):
        ones_v[j, pl.ds(c * _L, _L)] = one16
      return carry
    lax.fori_loop(0, _B, oloop, 0)
    plsc.subcore_barrier()

    def batch(b, carry):
      pltpu.sync_copy(dstw.at[pl.ds(w0 + b * _B, _B)], dst_v)
      pltpu.sync_copy(ones_v, cnt_sh.at[dst_v], add=True)
      return carry
    lax.fori_loop(0, _NB, batch, 0)

    plsc.subcore_barrier()
    pltpu.sync_copy(cnt_sh.at[pl.ds(r0, _RPT)],
                    cnt_out.at[cid, pl.ds(r0, _RPT)])

  return k


_cnt_kernel = _make_cnt()


def _matmul(x, w):
  n, d = x.shape
  m = w.shape[1]
  br = 1000

  def body(x_ref, w_ref, o_ref):
    o_ref[...] = jnp.dot(x_ref[...], w_ref[...],
                         preferred_element_type=jnp.float32)

  return pl.pallas_call(
      body,
      grid=(n // br,),
      in_specs=[pl.BlockSpec((br, d), lambda i: (i, 0)),
                pl.BlockSpec((d, m), lambda i: (0, 0))],
      out_specs=pl.BlockSpec((br, m), lambda i: (i, 0)),
      out_shape=jax.ShapeDtypeStruct((n, m), jnp.float32),
  )(x, w)


def _gauss(ea8, w1, w2, c):
  be = 8000

  def body(e_ref, w1_ref, w2_ref, c_ref, o_ref):
    e = e_ref[...]
    qa = jnp.dot(e * e, w2_ref[...], preferred_element_type=jnp.float32)
    qb = jnp.dot(e, w1_ref[...], preferred_element_type=jnp.float32)
    o_ref[...] = jnp.exp(qa + qb + c_ref[...])

  return pl.pallas_call(
      body,
      grid=(_E // be,),
      in_specs=[pl.BlockSpec((be, 8), lambda i: (i, 0)),
                pl.BlockSpec((8, 16), lambda i: (0, 0)),
                pl.BlockSpec((8, 16), lambda i: (0, 0)),
                pl.BlockSpec((1, 16), lambda i: (0, 0))],
      out_specs=pl.BlockSpec((be, 16), lambda i: (i, 0)),
      out_shape=jax.ShapeDtypeStruct((_E, 16), jnp.float32),
  )(ea8, w1, w2, c)


def _bn(y, gamma, beta):
  m = jnp.mean(y, axis=0, keepdims=True)
  v = jnp.mean((y - m) ** 2, axis=0, keepdims=True)
  return (y - m) * lax.rsqrt(v + 1e-5) * gamma + beta


def _elu(y):
  return jnp.where(y > 0, y, jnp.exp(jnp.minimum(y, 0.0)) - 1.0)


def _full(s):
  return pl.BlockSpec(s, lambda: (0,) * len(s))


def _bn_combine(mode, p0, p1, c0, c1, xr, b, g, be, extra=None):
  nd = (_N, _D)

  def body(*refs):
    if mode == "res":
      p0_r, p1_r, c0_r, c1_r, xr_r, b_r, g_r, be_r, ex_r, o_ref = refs
    else:
      p0_r, p1_r, c0_r, c1_r, xr_r, b_r, g_r, be_r, o_ref = refs
    cnt = jnp.clip(c0_r[...][:, :1] + c1_r[...][:, :1], 1.0)
    y = (p0_r[...] + p1_r[...]) / cnt + xr_r[...] + b_r[...]
    z = _bn(y, g_r[...], be_r[...])
    if mode == "elu":
      o_ref[...] = _elu(z)
    elif mode == "plain":
      o_ref[...] = z
    else:
      o_ref[...] = _elu(z + ex_r[...])

  in_specs = [_full(nd), _full(nd), _full((_N, _L)), _full((_N, _L)),
              _full(nd), _full((1, _D)), _full((1, _D)), _full((1, _D))]
  args = [p0, p1, c0, c1, xr, b, g, be]
  if mode == "res":
    in_specs.append(_full(nd))
    args.append(extra)
  return pl.pallas_call(
      body,
      in_specs=in_specs,
      out_specs=_full(nd),
      out_shape=jax.ShapeDtypeStruct(nd, jnp.float32),
  )(*args)


def kernel(x, edge_index, edge_attr, g1, mu1, sigma1, root1, b1,
           g2, mu2, sigma2, root2, b2, gs, mus, sigmas, roots, bs,
           bn1_g, bn1_b, bn2_g, bn2_b, bns_g, bns_b):
  f32 = jnp.float32
  src = edge_index[0]
  dst = edge_index[1]

  mu_all = jnp.concatenate([mu1, mu2, mus, jnp.zeros((5, 3), f32)], axis=0)
  sig_all = jnp.concatenate([sigma1, sigma2, sigmas, jnp.ones((5, 3), f32)],
                            axis=0)
  s2 = 1e-14 + sig_all ** 2
  w2 = jnp.concatenate([(-0.5 / s2).T, jnp.zeros((5, 16), f32)], axis=0)
  w1 = jnp.concatenate([(mu_all / s2).T, jnp.zeros((5, 16), f32)], axis=0)
  cq = (-0.5 * mu_all ** 2 / s2).sum(axis=1).reshape(1, 16)
  ea8 = jnp.concatenate([edge_attr, jnp.zeros((_E, 5), f32)], axis=1)
  gauss_all = _gauss(ea8, w1, w2, cq)

  wcat = jnp.concatenate([g1, gs, root1, roots], axis=1)
  xcat = _matmul(x, wcat)
  xt1, xts = xcat[:, : _K * _D], xcat[:, _K * _D: _K * _D + _D]
  xr1, xrs = xcat[:, 768:896], xcat[:, 896:1024]

  agg1 = _edge_agg_k5a(xt1, src, dst, gauss_all)
  aggs = _edge_agg_k1(xts, src, dst, gauss_all)
  cnt = _cnt_kernel(dst)

  c0, c1 = cnt[0, :_N, : _L], cnt[1, :_N, : _L]
  row = lambda v: v.reshape(1, _D)
  h = _bn_combine("elu", agg1[0, :_N], agg1[1, :_N], c0, c1,
                  xr1, row(b1), row(bn1_g), row(bn1_b))
  scbn = _bn_combine("plain", aggs[0, :_N], aggs[1, :_N], c0, c1,
                     xrs, row(bs), row(bns_g), row(bns_b))

  w2cat = jnp.concatenate([g2, root2], axis=1)
  hcat = _matmul(h, w2cat)
  xt2, xr2 = hcat[:, : _K * _D], hcat[:, _K * _D:]

  agg2 = _edge_agg_k5b(xt2, src, dst, gauss_all)

  return _bn_combine("res", agg2[0, :_N], agg2[1, :_N], c0, c1, xr2,
                     row(b2), row(bn2_g), row(bn2_b), extra=scbn)


# speedup vs baseline: 3.7523x; 1.8217x over previous
import functools

import jax
import jax.numpy as jnp
from jax import lax
from jax.experimental import pallas as pl
from jax.experimental.pallas import tpu as pltpu
from jax.experimental.pallas import tpu_sc as plsc

_N = 10000
_E = 320000
_D = 128
_K = 5
_NC, _NS, _L = 2, 16, 16
_NW = _NC * _NS
_EPT = _E // _NW
_B = 40
_NB = _EPT // _B
_NPAD = 10112
_RPT = _NPAD // _NS
_CH = _D // _L


_H0, _H1 = 24, 16


def _make_edge_agg(k_eff: int, k0: int):
  tw = k_eff * _D
  mesh = plsc.VectorSubcoreMesh(core_axis_name="c", subcore_axis_name="s")
  out_type = jax.ShapeDtypeStruct((_NC, _NPAD, _D), jnp.float32)
  scratch = [
      pltpu.VMEM_SHARED((_NPAD, _D), jnp.float32),
      pltpu.VMEM((2, _H0), jnp.int32),
      pltpu.VMEM((2, _H1), jnp.int32),
      pltpu.VMEM((2, _H0), jnp.int32),
      pltpu.VMEM((2, _H1), jnp.int32),
      pltpu.VMEM((2, _B, _L), jnp.float32),
      pltpu.VMEM((_B, tw), jnp.float32),
      pltpu.VMEM((_B, _D), jnp.float32),
      pltpu.SemaphoreType.DMA,
      pltpu.SemaphoreType.DMA,
      pltpu.SemaphoreType.DMA,
      pltpu.SemaphoreType.DMA,
      pltpu.SemaphoreType.DMA,
  ]

  @functools.partial(pl.kernel, out_type=out_type, mesh=mesh,
                     scratch_types=tuple(scratch))
  def k(table, srcw, dstw, gsub, agg_out, agg_sh, src0_v, src1_v,
        dst0_v, dst1_v, gs_v, rows_v, msg_v, st, sg0, sg1, ss0, ss1):
    cid = lax.axis_index("c")
    sid = lax.axis_index("s")
    w0 = (cid * _NS + sid) * _EPT
    r0 = sid * _RPT

    zero16 = jnp.zeros((_L,), jnp.float32)

    def zmsg(j, carry):
      for c in range(_CH):
        msg_v[j, pl.ds(c * _L, _L)] = zero16
      return carry
    lax.fori_loop(0, _B, zmsg, 0)
    nfull, rem = _RPT // _B, _RPT % _B
    for r in range(nfull):
      pltpu.sync_copy(msg_v, agg_sh.at[pl.ds(r0 + r * _B, _B)])
    if rem:
      pltpu.sync_copy(msg_v.at[pl.ds(0, rem)],
                      agg_sh.at[pl.ds(r0 + nfull * _B, rem)])
    plsc.subcore_barrier()

    rows0, rows1 = rows_v.at[pl.ds(0, _H0)], rows_v.at[pl.ds(_H0, _H1)]
    msg0, msg1 = msg_v.at[pl.ds(0, _H0)], msg_v.at[pl.ds(_H0, _H1)]

    def stage(b, slot, sem):
      e0 = w0 + b * _B
      return [
          pltpu.async_copy(srcw.at[pl.ds(e0, _H0)], src0_v.at[slot], sem),
          pltpu.async_copy(srcw.at[pl.ds(e0 + _H0, _H1)], src1_v.at[slot],
                           sem),
          pltpu.async_copy(dstw.at[pl.ds(e0, _H0)], dst0_v.at[slot], sem),
          pltpu.async_copy(dstw.at[pl.ds(e0 + _H0, _H1)], dst1_v.at[slot],
                           sem),
          pltpu.async_copy(gsub.at[pl.ds(e0, _B)], gs_v.at[slot], sem),
      ]

    def wait_g0():
      pltpu.make_async_copy(table.at[pl.ds(0, _H0)], rows0, sg0).wait()
    def wait_g1():
      pltpu.make_async_copy(table.at[pl.ds(0, _H1)], rows1, sg1).wait()
    def wait_s0():
      pltpu.make_async_copy(agg_out.at[0, pl.ds(0, _H0)], msg0, ss0).wait()
    def wait_s1():
      pltpu.make_async_copy(agg_out.at[0, pl.ds(0, _H1)], msg1, ss1).wait()

    def compute(slot, h):
      n_e, j0 = (_H0, 0) if h == 0 else (_H1, _H0)
      def edge(j, ecarry):
        gvec = gs_v[slot, j0 + j, pl.ds(0, _L)]
        g = [gvec[k0 + k] for k in range(k_eff)]
        for c in range(_CH):
          acc = g[0] * rows_v[j0 + j, pl.ds(c * _L, _L)]
          for kk in range(1, k_eff):
            acc = acc + g[kk] * rows_v[j0 + j, pl.ds(kk * _D + c * _L, _L)]
          msg_v[j0 + j, pl.ds(c * _L, _L)] = acc
        return ecarry
      lax.fori_loop(0, n_e, edge, 0)

    for h in stage(0, 0, st):
      h.wait()
    pltpu.async_copy(msg0, agg_sh.at[dst0_v.at[0]], ss0, add=True)
    pltpu.async_copy(msg1, agg_sh.at[dst1_v.at[0]], ss1, add=True)
    pltpu.async_copy(table.at[src0_v.at[0]], rows0, sg0)
    pltpu.async_copy(table.at[src1_v.at[0]], rows1, sg1)

    def pair(i, carry):
      for slot in (0, 1):
        b = 2 * i + slot
        wait_s0()
        wait_s1()
        nxt = slot ^ 1
        stages = stage((b + 1) % _NB, nxt, st)
        wait_g0()
        compute(slot, 0)
        pltpu.async_copy(msg0, agg_sh.at[dst0_v.at[slot]], ss0, add=True)
        for hh in stages:
          hh.wait()
        pltpu.async_copy(table.at[src0_v.at[nxt]], rows0, sg0)
        wait_g1()
        compute(slot, 1)
        pltpu.async_copy(msg1, agg_sh.at[dst1_v.at[slot]], ss1, add=True)
        pltpu.async_copy(table.at[src1_v.at[nxt]], rows1, sg1)
      return carry
    lax.fori_loop(0, _NB // 2, pair, 0)

    wait_s0()
    wait_s1()
    wait_g0()
    wait_g1()

    plsc.subcore_barrier()
    pltpu.sync_copy(agg_sh.at[pl.ds(r0, _RPT)],
                    agg_out.at[cid, pl.ds(r0, _RPT)])

  return k


_edge_agg_k5a = _make_edge_agg(_K, 0)
_edge_agg_k5b = _make_edge_agg(_K, _K)
_edge_agg_k1 = _make_edge_agg(1, 2 * _K)


def _make_cnt():
  mesh = plsc.VectorSubcoreMesh(core_axis_name="c", subcore_axis_name="s")
  out_type = jax.ShapeDtypeStruct((_NC, _NPAD, _D), jnp.float32)
  scratch = [
      pltpu.VMEM_SHARED((_NPAD, _D), jnp.float32),
      pltpu.VMEM((_B,), jnp.int32),
      pltpu.VMEM((_B, _D), jnp.float32),
  ]

  @functools.partial(pl.kernel, out_type=out_type, mesh=mesh,
                     scratch_types=tuple(scratch))
  def k(dstw, cnt_out, cnt_sh, dst_v, ones_v):
    cid = lax.axis_index("c")
    sid = lax.axis_index("s")
    w0 = (cid * _NS + sid) * _EPT
    r0 = sid * _RPT

    zero16 = jnp.zeros((_L,), jnp.float32)
    def zloop(j, carry):
      for c in range(_CH):
        ones_v[j, pl.ds(c * _L, _L)] = zero16
      return carry
    lax.fori_loop(0, _B, zloop, 0)
    nfull, rem = _RPT // _B, _RPT % _B
    for r in range(nfull):
      pltpu.sync_copy(ones_v, cnt_sh.at[pl.ds(r0 + r * _B, _B)])
    if rem:
      pltpu.sync_copy(ones_v.at[pl.ds(0, rem)],
                      cnt_sh.at[pl.ds(r0 + nfull * _B, rem)])
    one16 = jnp.ones((_L,), jnp.float32)
    def oloop(j, carry):
      for c in range(_CH):
        ones_v[j, pl.ds(c * _L, _L)] = one16
      return carry
    lax.fori_loop(0, _B, oloop, 0)
    plsc.subcore_barrier()

    def batch(b, carry):
      pltpu.sync_copy(dstw.at[pl.ds(w0 + b * _B, _B)], dst_v)
      pltpu.sync_copy(ones_v, cnt_sh.at[dst_v], add=True)
      return carry
    lax.fori_loop(0, _NB, batch, 0)

    plsc.subcore_barrier()
    pltpu.sync_copy(cnt_sh.at[pl.ds(r0, _RPT)],
                    cnt_out.at[cid, pl.ds(r0, _RPT)])

  return k


_cnt_kernel = _make_cnt()


def _matmul(x, w):
  n, d = x.shape
  m = w.shape[1]
  br = 1000

  def body(x_ref, w_ref, o_ref):
    o_ref[...] = jnp.dot(x_ref[...], w_ref[...],
                         preferred_element_type=jnp.float32)

  return pl.pallas_call(
      body,
      grid=(n // br,),
      in_specs=[pl.BlockSpec((br, d), lambda i: (i, 0)),
                pl.BlockSpec((d, m), lambda i: (0, 0))],
      out_specs=pl.BlockSpec((br, m), lambda i: (i, 0)),
      out_shape=jax.ShapeDtypeStruct((n, m), jnp.float32),
  )(x, w)


def _gauss(ea8, w1, w2, c):
  be = 8000

  def body(e_ref, w1_ref, w2_ref, c_ref, o_ref):
    e = e_ref[...]
    qa = jnp.dot(e * e, w2_ref[...], preferred_element_type=jnp.float32)
    qb = jnp.dot(e, w1_ref[...], preferred_element_type=jnp.float32)
    o_ref[...] = jnp.exp(qa + qb + c_ref[...])

  return pl.pallas_call(
      body,
      grid=(_E // be,),
      in_specs=[pl.BlockSpec((be, 8), lambda i: (i, 0)),
                pl.BlockSpec((8, 16), lambda i: (0, 0)),
                pl.BlockSpec((8, 16), lambda i: (0, 0)),
                pl.BlockSpec((1, 16), lambda i: (0, 0))],
      out_specs=pl.BlockSpec((be, 16), lambda i: (i, 0)),
      out_shape=jax.ShapeDtypeStruct((_E, 16), jnp.float32),
  )(ea8, w1, w2, c)


def _bn(y, gamma, beta):
  m = jnp.mean(y, axis=0, keepdims=True)
  v = jnp.mean((y - m) ** 2, axis=0, keepdims=True)
  return (y - m) * lax.rsqrt(v + 1e-5) * gamma + beta


def _elu(y):
  return jnp.where(y > 0, y, jnp.exp(jnp.minimum(y, 0.0)) - 1.0)


def _full(s):
  return pl.BlockSpec(s, lambda: (0,) * len(s))


def _bn_combine(mode, p0, p1, c0, c1, xr, b, g, be, extra=None):
  nd = (_N, _D)

  def body(*refs):
    if mode == "res":
      p0_r, p1_r, c0_r, c1_r, xr_r, b_r, g_r, be_r, ex_r, o_ref = refs
    else:
      p0_r, p1_r, c0_r, c1_r, xr_r, b_r, g_r, be_r, o_ref = refs
    cnt = jnp.clip(c0_r[...][:, :1] + c1_r[...][:, :1], 1.0)
    y = (p0_r[...] + p1_r[...]) / cnt + xr_r[...] + b_r[...]
    z = _bn(y, g_r[...], be_r[...])
    if mode == "elu":
      o_ref[...] = _elu(z)
    elif mode == "plain":
      o_ref[...] = z
    else:
      o_ref[...] = _elu(z + ex_r[...])

  in_specs = [_full(nd), _full(nd), _full((_N, _L)), _full((_N, _L)),
              _full(nd), _full((1, _D)), _full((1, _D)), _full((1, _D))]
  args = [p0, p1, c0, c1, xr, b, g, be]
  if mode == "res":
    in_specs.append(_full(nd))
    args.append(extra)
  return pl.pallas_call(
      body,
      in_specs=in_specs,
      out_specs=_full(nd),
      out_shape=jax.ShapeDtypeStruct(nd, jnp.float32),
  )(*args)


def kernel(x, edge_index, edge_attr, g1, mu1, sigma1, root1, b1,
           g2, mu2, sigma2, root2, b2, gs, mus, sigmas, roots, bs,
           bn1_g, bn1_b, bn2_g, bn2_b, bns_g, bns_b):
  f32 = jnp.float32
  src = edge_index[0]
  dst = edge_index[1]

  mu_all = jnp.concatenate([mu1, mu2, mus, jnp.zeros((5, 3), f32)], axis=0)
  sig_all = jnp.concatenate([sigma1, sigma2, sigmas, jnp.ones((5, 3), f32)],
                            axis=0)
  s2 = 1e-14 + sig_all ** 2
  w2 = jnp.concatenate([(-0.5 / s2).T, jnp.zeros((5, 16), f32)], axis=0)
  w1 = jnp.concatenate([(mu_all / s2).T, jnp.zeros((5, 16), f32)], axis=0)
  cq = (-0.5 * mu_all ** 2 / s2).sum(axis=1).reshape(1, 16)
  ea8 = jnp.concatenate([edge_attr, jnp.zeros((_E, 5), f32)], axis=1)
  gauss_all = _gauss(ea8, w1, w2, cq)

  wcat = jnp.concatenate([g1, gs, root1, roots], axis=1)
  xcat = _matmul(x, wcat)
  xt1, xts = xcat[:, : _K * _D], xcat[:, _K * _D: _K * _D + _D]
  xr1, xrs = xcat[:, 768:896], xcat[:, 896:1024]

  agg1 = _edge_agg_k5a(xt1, src, dst, gauss_all)
  aggs = _edge_agg_k1(xts, src, dst, gauss_all)
  cnt = _cnt_kernel(dst)

  c0, c1 = cnt[0, :_N, : _L], cnt[1, :_N, : _L]
  row = lambda v: v.reshape(1, _D)
  h = _bn_combine("elu", agg1[0, :_N], agg1[1, :_N], c0, c1,
                  xr1, row(b1), row(bn1_g), row(bn1_b))
  scbn = _bn_combine("plain", aggs[0, :_N], aggs[1, :_N], c0, c1,
                     xrs, row(bs), row(bns_g), row(bns_b))

  w2cat = jnp.concatenate([g2, root2], axis=1)
  hcat = _matmul(h, w2cat)
  xt2, xr2 = hcat[:, : _K * _D], hcat[:, _K * _D:]

  agg2 = _edge_agg_k5b(xt2, src, dst, gauss_all)

  return _bn_combine("res", agg2[0, :_N], agg2[1, :_N], c0, c1, xr2,
                     row(b2), row(bn2_g), row(bn2_b), extra=scbn)
